# baseline (speedup 1.0000x reference)
import functools

import jax
import jax.numpy as jnp
from jax import lax
from jax.experimental import pallas as pl
from jax.experimental.pallas import tpu as pltpu
from jax.experimental.pallas import tpu_sc as plsc

_NC, _NS = 2, 16
_NW = _NC * _NS
_RING = 5
_LANES = 128


@functools.cache
def _make_kernel(bsz: int, h: int, d: int):
    assert bsz == _NW * _LANES and d % 8 == 0 and h % _RING == 0
    dr = d // 8
    mesh = plsc.VectorSubcoreMesh(
        core_axis_name="c", subcore_axis_name="s",
        num_cores=_NC, num_subcores=_NS,
    )

    @functools.partial(
        pl.kernel,
        out_type=jax.ShapeDtypeStruct((h, dr, _NW, 8, _LANES), jnp.float32),
        mesh=mesh,
        scratch_types=[
            pltpu.VMEM((h, _LANES), jnp.int32),
            pltpu.VMEM((_RING, _LANES, d), jnp.float32),
            pltpu.VMEM((_RING, dr, 8, _LANES), jnp.float32),
        ] + [pltpu.SemaphoreType.DMA] * (2 * _RING),
        compiler_params=pltpu.CompilerParams(
            use_tc_tiling_on_sc=False, needs_layout_passes=False),
    )
    def k(idx_hbm, table_hbm, out_hbm, idx_v, gbuf, tbuf, *sems):
        gsem, wsem = sems[:_RING], sems[_RING:]
        wid = lax.axis_index("s") * _NC + lax.axis_index("c")
        pltpu.sync_copy(idx_hbm.at[:, pl.ds(wid * _LANES, _LANES)], idx_v)

        lane = lax.iota(jnp.int32, 16)
        rows = [lane + 16 * kk for kk in range(8)]

        for b in range(_RING):
            pltpu.async_copy(table_hbm.at[idx_v.at[b]], gbuf.at[b], gsem[b])

        @pl.loop(0, h // _RING)
        def _(j0):
            for b in range(_RING):
                j = j0 * _RING + b
                pltpu.make_async_copy(
                    table_hbm.at[idx_v.at[j]], gbuf.at[b], gsem[b]).wait()

                @pl.when(j0 > 0)
                def _():
                    pltpu.make_async_copy(
                        tbuf.at[b], out_hbm.at[j, :, wid], wsem[b]).wait()

                gb, tb = gbuf.at[b], tbuf.at[b]

                @pl.loop(0, dr)
                def _(fr):
                    for fs in range(8):
                        col = jnp.full((16,), 8 * fr + fs, jnp.int32)
                        for kk in range(8):
                            v = plsc.load_gather(gb, [rows[kk], col])
                            tb[fr, fs, pl.ds(16 * kk, 16)] = v

                pltpu.async_copy(tbuf.at[b], out_hbm.at[j, :, wid], wsem[b])
                j2 = j + _RING

                @pl.when(j2 < h)
                def _():
                    pltpu.async_copy(
                        table_hbm.at[idx_v.at[j2]], gbuf.at[b], gsem[b])

        for b in range(_RING):
            j = h - _RING + b
            pltpu.make_async_copy(
                tbuf.at[b], out_hbm.at[j, :, wid], wsem[b]).wait()

    return k


def kernel(indices, table):
    b, h = indices.shape
    v, d = table.shape
    table_p = jnp.pad(table, ((0, 0), (0, 128 - d))).reshape(v * 2, d)
    idx2 = (indices.astype(jnp.int32) * 2).T
    t = _make_kernel(b, h, d)(idx2, table_p)
    return t.transpose(2, 4, 0, 1, 3).reshape(b, h, d)


# speedup vs baseline: 1.2415x; 1.2415x over previous
import functools

import jax
import jax.numpy as jnp
from jax import lax
from jax.experimental import pallas as pl
from jax.experimental.pallas import tpu as pltpu
from jax.experimental.pallas import tpu_sc as plsc

_NC, _NS = 2, 16
_NW = _NC * _NS
_RING = 5
_LANES = 128


@functools.cache
def _make_kernel(bsz: int, h: int, d: int):
    assert bsz == _NW * _LANES and d % 8 == 0 and h % _RING == 0
    dr = d // 8
    mesh = plsc.VectorSubcoreMesh(
        core_axis_name="c", subcore_axis_name="s",
        num_cores=_NC, num_subcores=_NS,
    )

    @functools.partial(
        pl.kernel,
        out_type=jax.ShapeDtypeStruct((h, dr, _NW, 8, _LANES), jnp.float32),
        mesh=mesh,
        scratch_types=[
            pltpu.VMEM((h, _LANES), jnp.int32),
            pltpu.VMEM((_RING, _LANES, d), jnp.float32),
            pltpu.VMEM((_RING, dr, 8, _LANES), jnp.float32),
        ] + [pltpu.SemaphoreType.DMA] * (2 * _RING),
        compiler_params=pltpu.CompilerParams(
            use_tc_tiling_on_sc=False, needs_layout_passes=False),
    )
    def k(idx_hbm, table_hbm, out_hbm, idx_v, gbuf, tbuf, *sems):
        gsem, wsem = sems[:_RING], sems[_RING:]
        wid = lax.axis_index("s") * _NC + lax.axis_index("c")
        pltpu.sync_copy(idx_hbm.at[:, pl.ds(wid * _LANES, _LANES)], idx_v)

        lane = lax.iota(jnp.int32, 16)
        rows = [lane + 16 * kk for kk in range(8)]

        for b in range(_RING):
            pltpu.async_copy(table_hbm.at[idx_v.at[b]], gbuf.at[b], gsem[b])

        @pl.loop(0, h // _RING)
        def _(j0):
            for b in range(_RING):
                j = j0 * _RING + b
                pltpu.make_async_copy(
                    table_hbm.at[idx_v.at[j]], gbuf.at[b], gsem[b]).wait()

                @pl.when(j0 > 0)
                def _():
                    pltpu.make_async_copy(
                        tbuf.at[b], out_hbm.at[j, :, wid], wsem[b]).wait()

                gb, tb = gbuf.at[b], tbuf.at[b]

                @pl.loop(0, dr)
                def _(fr):
                    for fs in range(0, 8, 2):
                        cols = [jnp.full((16,), 8 * fr + fs + i, jnp.int32)
                                for i in range(2)]
                        vs = [plsc.load_gather(gb, [rows[kk], cols[i]])
                              for i in range(2) for kk in range(8)]
                        for i in range(2):
                            for kk in range(8):
                                tb[fr, fs + i, pl.ds(16 * kk, 16)] = vs[8 * i + kk]

                pltpu.async_copy(tbuf.at[b], out_hbm.at[j, :, wid], wsem[b])
                j2 = j + _RING

                @pl.when(j2 < h)
                def _():
                    pltpu.async_copy(
                        table_hbm.at[idx_v.at[j2]], gbuf.at[b], gsem[b])

        for b in range(_RING):
            j = h - _RING + b
            pltpu.make_async_copy(
                tbuf.at[b], out_hbm.at[j, :, wid], wsem[b]).wait()

    return k


def kernel(indices, table):
    b, h = indices.shape
    v, d = table.shape
    table_p = jnp.pad(table, ((0, 0), (0, 128 - d))).reshape(v * 2, d)
    idx2 = (indices.astype(jnp.int32) * 2).T
    t = _make_kernel(b, h, d)(idx2, table_p)
    return t.transpose(2, 4, 0, 1, 3).reshape(b, h, d)


# speedup vs baseline: 1.5129x; 1.2186x over previous
import functools

import jax
import jax.numpy as jnp
from jax import lax
from jax.experimental import pallas as pl
from jax.experimental.pallas import tpu as pltpu
from jax.experimental.pallas import tpu_sc as plsc

_NC, _NS = 2, 16
_NW = _NC * _NS
_RING = 8


@functools.cache
def _make_kernel(bsz: int, h: int, d: int):
    rows_per_w = bsz // _NW
    assert rows_per_w % _RING == 0
    mesh = plsc.VectorSubcoreMesh(
        core_axis_name="c", subcore_axis_name="s",
        num_cores=_NC, num_subcores=_NS,
    )

    @functools.partial(
        pl.kernel,
        out_type=jax.ShapeDtypeStruct((bsz, h, 128), jnp.float32),
        mesh=mesh,
        scratch_types=[
            pltpu.VMEM((rows_per_w, h), jnp.int32),
            pltpu.VMEM((_RING, h, d), jnp.float32),
        ] + [pltpu.SemaphoreType.DMA] * (2 * _RING),
        compiler_params=pltpu.CompilerParams(use_tc_tiling_on_sc=False),
    )
    def k(idx_hbm, table_hbm, out_hbm, idx_v, bufs, *sems):
        gsem, wsem = sems[:_RING], sems[_RING:]
        wid = lax.axis_index("s") * _NC + lax.axis_index("c")
        row0 = wid * rows_per_w
        pltpu.sync_copy(idx_hbm.at[pl.ds(row0, rows_per_w)], idx_v)

        for b in range(_RING):
            pltpu.async_copy(table_hbm.at[idx_v.at[b]], bufs.at[b], gsem[b])

        @pl.loop(0, rows_per_w, step=_RING)
        def _(j0):
            for b in range(_RING):
                j = j0 + b
                pltpu.make_async_copy(
                    table_hbm.at[idx_v.at[j]], bufs.at[b], gsem[b]).wait()
                pltpu.async_copy(
                    bufs.at[b], out_hbm.at[row0 + j, :, pl.ds(0, d)], wsem[b])
                j2 = j + _RING

                @pl.when(j2 < rows_per_w)
                def _():
                    pltpu.make_async_copy(
                        bufs.at[b], out_hbm.at[row0 + j, :, pl.ds(0, d)],
                        wsem[b]).wait()
                    pltpu.async_copy(
                        table_hbm.at[idx_v.at[j2]], bufs.at[b], gsem[b])

        for b in range(_RING):
            j = rows_per_w - _RING + b
            pltpu.make_async_copy(
                bufs.at[b], out_hbm.at[row0 + j, :, pl.ds(0, d)],
                wsem[b]).wait()

    return k


def kernel(indices, table):
    b, h = indices.shape
    v, d = table.shape
    table_p = jnp.pad(table, ((0, 0), (0, 128 - d))).reshape(v * 2, d)
    idx2 = indices.astype(jnp.int32) * 2
    out = _make_kernel(b, h, d)(idx2, table_p)
    return out[:, :, :d]


# speedup vs baseline: 1.6535x; 1.0930x over previous
import functools

import jax
import jax.numpy as jnp
from jax import lax
from jax.experimental import pallas as pl
from jax.experimental.pallas import tpu as pltpu
from jax.experimental.pallas import tpu_sc as plsc

_NC, _NS = 2, 16
_NW = _NC * _NS
_RING = 8


@functools.cache
def _make_kernel(bsz: int, h: int, d: int):
    rows_per_w = bsz // _NW
    assert rows_per_w % _RING == 0
    mesh = plsc.VectorSubcoreMesh(
        core_axis_name="c", subcore_axis_name="s",
        num_cores=_NC, num_subcores=_NS,
    )

    @functools.partial(
        pl.kernel,
        out_type=jax.ShapeDtypeStruct((bsz, h, d), jnp.float32),
        mesh=mesh,
        scratch_types=[
            pltpu.VMEM((rows_per_w, h), jnp.int32),
            pltpu.VMEM((_RING, h, d), jnp.float32),
        ] + [pltpu.SemaphoreType.DMA] * (2 * _RING),
        compiler_params=pltpu.CompilerParams(use_tc_tiling_on_sc=False),
    )
    def k(idx_hbm, table_hbm, out_hbm, idx_v, bufs, *sems):
        gsem, wsem = sems[:_RING], sems[_RING:]
        wid = lax.axis_index("s") * _NC + lax.axis_index("c")
        row0 = wid * rows_per_w
        pltpu.sync_copy(idx_hbm.at[pl.ds(row0, rows_per_w)], idx_v)

        for b in range(_RING):
            pltpu.async_copy(table_hbm.at[idx_v.at[b]], bufs.at[b], gsem[b])

        @pl.loop(0, rows_per_w, step=_RING)
        def _(j0):
            for b in range(_RING):
                j = j0 + b
                pltpu.make_async_copy(
                    table_hbm.at[idx_v.at[j]], bufs.at[b], gsem[b]).wait()
                pltpu.async_copy(
                    bufs.at[b], out_hbm.at[row0 + j], wsem[b])
                j2 = j + _RING

                @pl.when(j2 < rows_per_w)
                def _():
                    pltpu.make_async_copy(
                        bufs.at[b], out_hbm.at[row0 + j], wsem[b]).wait()
                    pltpu.async_copy(
                        table_hbm.at[idx_v.at[j2]], bufs.at[b], gsem[b])

        for b in range(_RING):
            j = rows_per_w - _RING + b
            pltpu.make_async_copy(
                bufs.at[b], out_hbm.at[row0 + j], wsem[b]).wait()

    return k


def kernel(indices, table):
    b, h = indices.shape
    v, d = table.shape
    table_p = jnp.pad(table, ((0, 0), (0, 128 - d))).reshape(v * 2, d)
    idx2 = indices.astype(jnp.int32) * 2
    return _make_kernel(b, h, d)(idx2, table_p)
